# baseline (device time: 88021 ns/iter reference)
import jax
import jax.numpy as jnp
from jax import lax
from jax.experimental import pallas as pl
from jax.experimental.pallas import tpu as pltpu

N_DEV = 4
N_ROUNDS = 4
N_SUB = 2


def _gelu(y):
    c = 0.7978845608028654
    return 0.5 * y * (1.0 + jnp.tanh(c * (y + 0.044715 * y ** 3)))


def kernel(x, w_mat):
    m, k_per = x.shape
    _, n = w_mat.shape
    chunk = m // N_DEV
    n2 = n // 2
    sc = n2 // N_SUB

    def body(x_ref, w_ref, out_ref, wbf_ref,
             sb1A, rb1A, sb2A, rb2A, ksA,
             sb1B, rb1B, sb2B, rb2B, ksB,
             ssA, rsA, ssB, rsB):
        my = lax.axis_index("i")
        p1 = jnp.bitwise_xor(my, 1)
        p2 = 3 - my

        barrier_sem = pltpu.get_barrier_semaphore()
        for nbr in (p1, p2):
            pl.semaphore_signal(
                barrier_sem, inc=1,
                device_id=(nbr,), device_id_type=pl.DeviceIdType.MESH,
            )
        pl.semaphore_wait(barrier_sem, 2)

        wbf_ref[...] = w_ref[...].astype(jnp.bfloat16)

        gbit = jnp.bitwise_and(jnp.bitwise_xor(my, my >> 1), 1)
        hbit = my >> 1
        b0 = jnp.bitwise_and(my, 1)

        schemes = [
            dict(c0=0, sb1=sb1A, rb1=rb1A, sb2=sb2A, rb2=rb2A, ks=ksA,
                 ss=ssA, rs=rsA, partners=[p1, p2, p2, p1],
                 keep1=gbit * (2 * chunk), so1=(1 - gbit) * (2 * chunk),
                 fo=gbit * (2 * chunk) + hbit * chunk,
                 so2=gbit * (2 * chunk) + (1 - hbit) * chunk),
            dict(c0=n2, sb1=sb1B, rb1=rb1B, sb2=sb2B, rb2=rb2B, ks=ksB,
                 ss=ssB, rs=rsB, partners=[p2, p1, p1, p2],
                 keep1=hbit * (2 * chunk), so1=(1 - hbit) * (2 * chunk),
                 fo=my * chunk,
                 so2=hbit * (2 * chunk) + (1 - b0) * chunk),
        ]

        def out_at(S, row_off, nrows, s):
            return out_ref.at[pl.ds(row_off, nrows), pl.ds(S["c0"] + s * sc, sc)]

        def mk(S, r, s, src, dst):
            return pltpu.make_async_remote_copy(
                src_ref=src, dst_ref=dst,
                send_sem=S["ss"].at[r, s],
                recv_sem=S["rs"].at[r, s],
                device_id=(S["partners"][r],),
                device_id_type=pl.DeviceIdType.MESH,
            )

        send = [
            [
                [
                    mk(S, 0, s, S["sb1"].at[:, pl.ds(s * sc, sc)],
                       S["rb1"].at[:, pl.ds(s * sc, sc)]),
                    mk(S, 1, s, S["sb2"].at[:, pl.ds(s * sc, sc)],
                       S["rb2"].at[:, pl.ds(s * sc, sc)]),
                    mk(S, 2, s, out_at(S, S["fo"], chunk, s),
                       out_at(S, S["fo"], chunk, s)),
                    mk(S, 3, s, out_at(S, S["keep1"], 2 * chunk, s),
                       out_at(S, S["keep1"], 2 * chunk, s)),
                ]
                for s in range(N_SUB)
            ]
            for S in schemes
        ]
        recv34 = [
            [
                [
                    mk(S, 2, s, out_at(S, S["so2"], chunk, s),
                       out_at(S, S["so2"], chunk, s)),
                    mk(S, 3, s, out_at(S, S["so1"], 2 * chunk, s),
                       out_at(S, S["so1"], 2 * chunk, s)),
                ]
                for s in range(N_SUB)
            ]
            for S in schemes
        ]

        def pdot(row_off, nrows, S, s):
            c = S["c0"] + s * sc
            return jnp.dot(
                x_ref[pl.ds(row_off, nrows), :].astype(jnp.bfloat16),
                wbf_ref[:, c:c + sc],
                preferred_element_type=jnp.float32,
            )

        for s in range(N_SUB):
            for i, S in enumerate(schemes):
                S["sb1"][:, s * sc:(s + 1) * sc] = (
                    pdot(S["so1"], 2 * chunk, S, s).astype(jnp.bfloat16)
                )
            for i in range(2):
                send[i][s][0].start()

        p_s2 = [[pdot(S["so2"], chunk, S, s) for s in range(N_SUB)]
                for S in schemes]
        p_k = [[pdot(S["fo"], chunk, S, s) for s in range(N_SUB)]
               for S in schemes]

        for s in range(N_SUB):
            for i in range(2):
                send[i][s][0].wait_recv()
            for i, S in enumerate(schemes):
                lso2 = S["so2"] - S["keep1"]
                S["sb2"][:, s * sc:(s + 1) * sc] = (
                    S["rb1"][pl.ds(lso2, chunk), s * sc:(s + 1) * sc]
                    .astype(jnp.float32) + p_s2[i][s]
                ).astype(jnp.bfloat16)
            for i in range(2):
                send[i][s][1].start()
            for i, S in enumerate(schemes):
                lfo = S["fo"] - S["keep1"]
                S["ks"][:, s * sc:(s + 1) * sc] = (
                    S["rb1"][pl.ds(lfo, chunk), s * sc:(s + 1) * sc]
                    .astype(jnp.float32) + p_k[i][s]
                )

        for s in range(N_SUB):
            for i in range(2):
                send[i][s][1].wait_recv()
            for i, S in enumerate(schemes):
                summed = (
                    S["rb2"][:, s * sc:(s + 1) * sc].astype(jnp.float32)
                    + S["ks"][:, s * sc:(s + 1) * sc]
                )
                g = _gelu(summed).astype(jnp.bfloat16)
                out_ref[pl.ds(S["fo"], chunk),
                        S["c0"] + s * sc:S["c0"] + (s + 1) * sc] = g
            for i in range(2):
                send[i][s][2].start()

        for s in range(N_SUB):
            for i in range(2):
                recv34[i][s][0].wait_recv()
            for i in range(2):
                send[i][s][3].start()
        for s in range(N_SUB):
            for i in range(2):
                recv34[i][s][1].wait_recv()

        for i in range(2):
            for s in range(N_SUB):
                for r in range(N_ROUNDS):
                    send[i][s][r].wait_send()

    two_chunk = 2 * (m // N_DEV)
    return pl.pallas_call(
        body,
        out_shape=jax.ShapeDtypeStruct((m, n), jnp.bfloat16),
        in_specs=[
            pl.BlockSpec(memory_space=pltpu.VMEM),
            pl.BlockSpec(memory_space=pltpu.VMEM),
        ],
        out_specs=pl.BlockSpec(memory_space=pltpu.VMEM),
        scratch_shapes=[
            pltpu.VMEM((k_per, n), jnp.bfloat16),
            pltpu.VMEM((two_chunk, n2), jnp.bfloat16),
            pltpu.VMEM((two_chunk, n2), jnp.bfloat16),
            pltpu.VMEM((m // N_DEV, n2), jnp.bfloat16),
            pltpu.VMEM((m // N_DEV, n2), jnp.bfloat16),
            pltpu.VMEM((m // N_DEV, n2), jnp.float32),
            pltpu.VMEM((two_chunk, n2), jnp.bfloat16),
            pltpu.VMEM((two_chunk, n2), jnp.bfloat16),
            pltpu.VMEM((m // N_DEV, n2), jnp.bfloat16),
            pltpu.VMEM((m // N_DEV, n2), jnp.bfloat16),
            pltpu.VMEM((m // N_DEV, n2), jnp.float32),
            pltpu.SemaphoreType.DMA((N_ROUNDS, N_SUB)),
            pltpu.SemaphoreType.DMA((N_ROUNDS, N_SUB)),
            pltpu.SemaphoreType.DMA((N_ROUNDS, N_SUB)),
            pltpu.SemaphoreType.DMA((N_ROUNDS, N_SUB)),
        ],
        compiler_params=pltpu.CompilerParams(
            collective_id=0,
            vmem_limit_bytes=100 * 1024 * 1024,
        ),
    )(x, w_mat)


# device time: 86884 ns/iter; 1.0131x vs baseline; 1.0131x over previous
import jax
import jax.numpy as jnp
from jax import lax
from jax.experimental import pallas as pl
from jax.experimental.pallas import tpu as pltpu

N_DEV = 4
N_ROUNDS = 4
N_SUB = 2


def _gelu(y):
    c = 0.7978845608028654
    return 0.5 * y * (1.0 + jnp.tanh(c * (y + 0.044715 * y ** 3)))


def kernel(x, w_mat):
    m, k_per = x.shape
    _, n = w_mat.shape
    chunk = m // N_DEV
    n2 = n // 2
    sc = n2 // N_SUB

    def body(x_ref, w_ref, out_ref, wbf_ref,
             sb1A, rb1A, sb2A, rb2A, ksA,
             sb1B, rb1B, sb2B, rb2B, ksB,
             ssA, rsA, ssB, rsB):
        my = lax.axis_index("i")
        p1 = jnp.bitwise_xor(my, 1)
        p2 = 3 - my

        barrier_sem = pltpu.get_barrier_semaphore()
        for nbr in (p1, p2):
            pl.semaphore_signal(
                barrier_sem, inc=1,
                device_id=(nbr,), device_id_type=pl.DeviceIdType.MESH,
            )

        gbit = jnp.bitwise_and(jnp.bitwise_xor(my, my >> 1), 1)
        hbit = my >> 1
        b0 = jnp.bitwise_and(my, 1)

        schemes = [
            dict(c0=0, sb1=sb1A, rb1=rb1A, sb2=sb2A, rb2=rb2A, ks=ksA,
                 ss=ssA, rs=rsA, partners=[p1, p2, p2, p1],
                 keep1=gbit * (2 * chunk), so1=(1 - gbit) * (2 * chunk),
                 fo=gbit * (2 * chunk) + hbit * chunk,
                 so2=gbit * (2 * chunk) + (1 - hbit) * chunk),
            dict(c0=n2, sb1=sb1B, rb1=rb1B, sb2=sb2B, rb2=rb2B, ks=ksB,
                 ss=ssB, rs=rsB, partners=[p2, p1, p1, p2],
                 keep1=hbit * (2 * chunk), so1=(1 - hbit) * (2 * chunk),
                 fo=my * chunk,
                 so2=hbit * (2 * chunk) + (1 - b0) * chunk),
        ]

        def out_at(S, row_off, nrows, s):
            return out_ref.at[pl.ds(row_off, nrows), pl.ds(S["c0"] + s * sc, sc)]

        def mk(S, r, s, src, dst):
            return pltpu.make_async_remote_copy(
                src_ref=src, dst_ref=dst,
                send_sem=S["ss"].at[r, s],
                recv_sem=S["rs"].at[r, s],
                device_id=(S["partners"][r],),
                device_id_type=pl.DeviceIdType.MESH,
            )

        send = [
            [
                [
                    mk(S, 0, s, S["sb1"].at[:, pl.ds(s * sc, sc)],
                       S["rb1"].at[:, pl.ds(s * sc, sc)]),
                    mk(S, 1, s, S["sb2"].at[:, pl.ds(s * sc, sc)],
                       S["rb2"].at[:, pl.ds(s * sc, sc)]),
                    mk(S, 2, s, out_at(S, S["fo"], chunk, s),
                       out_at(S, S["fo"], chunk, s)),
                    mk(S, 3, s, out_at(S, S["keep1"], 2 * chunk, s),
                       out_at(S, S["keep1"], 2 * chunk, s)),
                ]
                for s in range(N_SUB)
            ]
            for S in schemes
        ]
        recv34 = [
            [
                [
                    mk(S, 2, s, out_at(S, S["so2"], chunk, s),
                       out_at(S, S["so2"], chunk, s)),
                    mk(S, 3, s, out_at(S, S["so1"], 2 * chunk, s),
                       out_at(S, S["so1"], 2 * chunk, s)),
                ]
                for s in range(N_SUB)
            ]
            for S in schemes
        ]

        def pdot(row_off, nrows, S, s):
            c = S["c0"] + s * sc
            return jnp.dot(
                x_ref[pl.ds(row_off, nrows), :].astype(jnp.bfloat16),
                wbf_ref[:, c:c + sc],
                preferred_element_type=jnp.float32,
            )

        for s in range(N_SUB):
            for i, S in enumerate(schemes):
                c = S["c0"] + s * sc
                wbf_ref[:, c:c + sc] = w_ref[:, c:c + sc].astype(jnp.bfloat16)
                S["sb1"][:, s * sc:(s + 1) * sc] = (
                    pdot(S["so1"], 2 * chunk, S, s).astype(jnp.bfloat16)
                )
            if s == 0:
                pl.semaphore_wait(barrier_sem, 2)
            for i in range(2):
                send[i][s][0].start()

        p_s2 = [[pdot(S["so2"], chunk, S, s) for s in range(N_SUB)]
                for S in schemes]
        p_k = [[pdot(S["fo"], chunk, S, s) for s in range(N_SUB)]
               for S in schemes]

        for s in range(N_SUB):
            for i in range(2):
                send[i][s][0].wait_recv()
            for i, S in enumerate(schemes):
                lso2 = S["so2"] - S["keep1"]
                S["sb2"][:, s * sc:(s + 1) * sc] = (
                    S["rb1"][pl.ds(lso2, chunk), s * sc:(s + 1) * sc]
                    .astype(jnp.float32) + p_s2[i][s]
                ).astype(jnp.bfloat16)
            for i in range(2):
                send[i][s][1].start()
            for i, S in enumerate(schemes):
                lfo = S["fo"] - S["keep1"]
                S["ks"][:, s * sc:(s + 1) * sc] = (
                    S["rb1"][pl.ds(lfo, chunk), s * sc:(s + 1) * sc]
                    .astype(jnp.float32) + p_k[i][s]
                )

        for s in range(N_SUB):
            for i in range(2):
                send[i][s][1].wait_recv()
            for i, S in enumerate(schemes):
                summed = (
                    S["rb2"][:, s * sc:(s + 1) * sc].astype(jnp.float32)
                    + S["ks"][:, s * sc:(s + 1) * sc]
                )
                g = _gelu(summed).astype(jnp.bfloat16)
                out_ref[pl.ds(S["fo"], chunk),
                        S["c0"] + s * sc:S["c0"] + (s + 1) * sc] = g
            for i in range(2):
                send[i][s][2].start()

        for s in range(N_SUB):
            for i in range(2):
                recv34[i][s][0].wait_recv()
            for i in range(2):
                send[i][s][3].start()
        for s in range(N_SUB):
            for i in range(2):
                recv34[i][s][1].wait_recv()

        for i in range(2):
            for s in range(N_SUB):
                for r in range(N_ROUNDS):
                    send[i][s][r].wait_send()

    two_chunk = 2 * (m // N_DEV)
    return pl.pallas_call(
        body,
        out_shape=jax.ShapeDtypeStruct((m, n), jnp.bfloat16),
        in_specs=[
            pl.BlockSpec(memory_space=pltpu.VMEM),
            pl.BlockSpec(memory_space=pltpu.VMEM),
        ],
        out_specs=pl.BlockSpec(memory_space=pltpu.VMEM),
        scratch_shapes=[
            pltpu.VMEM((k_per, n), jnp.bfloat16),
            pltpu.VMEM((two_chunk, n2), jnp.bfloat16),
            pltpu.VMEM((two_chunk, n2), jnp.bfloat16),
            pltpu.VMEM((m // N_DEV, n2), jnp.bfloat16),
            pltpu.VMEM((m // N_DEV, n2), jnp.bfloat16),
            pltpu.VMEM((m // N_DEV, n2), jnp.float32),
            pltpu.VMEM((two_chunk, n2), jnp.bfloat16),
            pltpu.VMEM((two_chunk, n2), jnp.bfloat16),
            pltpu.VMEM((m // N_DEV, n2), jnp.bfloat16),
            pltpu.VMEM((m // N_DEV, n2), jnp.bfloat16),
            pltpu.VMEM((m // N_DEV, n2), jnp.float32),
            pltpu.SemaphoreType.DMA((N_ROUNDS, N_SUB)),
            pltpu.SemaphoreType.DMA((N_ROUNDS, N_SUB)),
            pltpu.SemaphoreType.DMA((N_ROUNDS, N_SUB)),
            pltpu.SemaphoreType.DMA((N_ROUNDS, N_SUB)),
        ],
        compiler_params=pltpu.CompilerParams(
            collective_id=0,
            vmem_limit_bytes=100 * 1024 * 1024,
        ),
    )(x, w_mat)
